# baseline (speedup 1.0000x reference)
import dataclasses
import functools

import jax
import jax.numpy as jnp
from jax import lax
from jax.experimental import pallas as pl
from jax.experimental.pallas import tpu as pltpu
from jax.experimental.pallas import tpu_sc as plsc

_EPS = 1e-15
_NC = 2
_NS = 16
_L = 16
_CH = 32
_G16 = 16
_WB = 32


def _pre_logmap(x):
    n, d = x.shape
    blk = 1000

    def body(x_ref, o_ref):
        xb = x_ref[...]
        pn = jnp.sqrt(jnp.sum(xb * xb, axis=1, keepdims=True))
        pn = jnp.maximum(pn, _EPS)
        z = jnp.clip(pn, -1.0 + 1e-7, 1.0 - 1e-7)
        at = 0.5 * (jnp.log1p(z) - jnp.log1p(-z))
        o_ref[...] = xb / pn * at

    return pl.pallas_call(
        body,
        grid=(n // blk,),
        in_specs=[pl.BlockSpec((blk, d), lambda i: (i, 0))],
        out_specs=pl.BlockSpec((blk, d), lambda i: (i, 0)),
        out_shape=jax.ShapeDtypeStruct((n, d), jnp.float32),
    )(x)


def _spmm_sc(pk, ng, xt, n, d, maxg):
    n2 = n // 2
    rpt = (n2 // _NS) // 8 * 8
    tail = n2 - _NS * rpt
    wfull = rpt // _WB
    wrem = rpt - wfull * _WB

    mesh = plsc.VectorSubcoreMesh(core_axis_name="c", subcore_axis_name="s")
    cp = pltpu.CompilerParams()
    if "needs_layout_passes" in pltpu.CompilerParams.__dataclass_fields__:
        cp = dataclasses.replace(cp, needs_layout_passes=False)

    hd = d // 2

    @functools.partial(
        pl.kernel,
        out_type=jax.ShapeDtypeStruct((n, 2, hd), jnp.float32),
        mesh=mesh,
        compiler_params=cp,
        scratch_types=[
            pltpu.VMEM_SHARED((n2, 2, hd), jnp.float32),
            pltpu.VMEM((_G16, 4, _CH), jnp.int32),
            pltpu.VMEM((4, _CH, 2, hd), jnp.float32),
            pltpu.SemaphoreType.DMA((4,)),
            pltpu.SemaphoreType.DMA((4,)),
        ],
    )
    def run(pk_hbm, ng_hbm, x_hbm, out_hbm, acc, ibuf, rows, gsem, ssem):
        c = lax.axis_index("c")
        s = lax.axis_index("s")
        base = s * rpt

        pltpu.sync_copy(ng_hbm, ibuf.at[0, 0, pl.ds(0, _L)])
        ngv = ibuf[0, 0, pl.ds(0, _L)]
        lanes = lax.broadcasted_iota(jnp.int32, (_L,), 0)
        ngc = jnp.max(jnp.where(lanes == c, ngv, 0))

        @pl.loop(0, _WB)
        def _(i):
            for h in range(2):
                for j in range(hd // _L):
                    rows[0, i, h, pl.ds(j * _L, _L)] = jnp.zeros(
                        (_L,), jnp.float32)

        for k in range(wfull):
            pltpu.sync_copy(rows.at[0], acc.at[pl.ds(base + k * _WB, _WB)])
        if wrem:
            pltpu.sync_copy(rows.at[0, pl.ds(0, wrem)],
                            acc.at[pl.ds(base + wfull * _WB, wrem)])
        if tail:
            @pl.when(s == _NS - 1)
            def _():
                pltpu.sync_copy(rows.at[0, pl.ds(0, tail)],
                                acc.at[pl.ds(_NS * rpt, tail)])
        plsc.subcore_barrier()

        def issue_gather(cc, b):
            pltpu.async_copy(x_hbm.at[ibuf.at[cc, 0]], rows.at[b],
                             gsem.at[b])

        def wait_gather(cc, b):
            pltpu.make_async_copy(x_hbm.at[ibuf.at[cc, 0]], rows.at[b],
                                  gsem.at[b]).wait()

        def wait_scatter(cc, b):
            pltpu.make_async_copy(rows.at[b], acc.at[ibuf.at[cc, 1]],
                                  ssem.at[b]).wait()

        @pl.loop(0, maxg)
        def _(g):
            @pl.when(g < ngc)
            def _():
                @pl.when(g > 0)
                def _():
                    for b in range(4):
                        wait_scatter(_G16 - 4 + b, b)

                pltpu.sync_copy(pk_hbm.at[c, s, g], ibuf)
                issue_gather(0, 0)
                issue_gather(1, 1)
                for cc in range(_G16):
                    b = cc % 4
                    if cc < _G16 - 2:
                        if cc >= 2:
                            wait_scatter(cc - 2, (cc + 2) % 4)
                        issue_gather(cc + 2, (cc + 2) % 4)
                    wait_gather(cc, b)

                    @pl.loop(0, _CH, step=2)
                    def _(e):
                        for u in range(2):
                            wv = plsc.bitcast(
                                plsc.load_gather(
                                    ibuf.at[cc, 2],
                                    [jnp.full((_L,), e + u, jnp.int32)]),
                                jnp.float32)
                            for h in range(2):
                                for j in range(hd // _L):
                                    sl = pl.ds(j * _L, _L)
                                    rows[b, e + u, h, sl] = (
                                        rows[b, e + u, h, sl] * wv)

                    pltpu.async_copy(rows.at[b], acc.at[ibuf.at[cc, 1]],
                                     ssem.at[b], add=True)

        @pl.when(ngc > 0)
        def _():
            for b in range(4):
                wait_scatter(_G16 - 4 + b, b)

        plsc.subcore_barrier()

        obase = c * n2 + base
        for k in range(wfull):
            pltpu.sync_copy(acc.at[pl.ds(base + k * _WB, _WB)],
                            out_hbm.at[pl.ds(obase + k * _WB, _WB)])
        if wrem:
            pltpu.sync_copy(acc.at[pl.ds(base + wfull * _WB, wrem)],
                            out_hbm.at[pl.ds(obase + wfull * _WB, wrem)])
        if tail:
            @pl.when(s == _NS - 1)
            def _():
                pltpu.sync_copy(acc.at[pl.ds(_NS * rpt, tail)],
                                out_hbm.at[pl.ds(c * n2 + _NS * rpt, tail)])

    return run(pk, ng, xt.reshape(n, 2, hd)).reshape(n, d)


def _post(y1, y2, y3, n, d):
    blk = 1000
    g = n // blk

    def body(y1_ref, y2_ref, y3_ref, o_ref, acc):
        p = pl.program_id(0)
        i = pl.program_id(1)
        sb = y1_ref[...] + y2_ref[...] + y3_ref[...]

        @pl.when(jnp.logical_and(p == 0, i == 0))
        def _():
            acc[...] = jnp.zeros_like(acc)

        @pl.when(p == 0)
        def _():
            acc[...] += jnp.sum(sb, axis=0, keepdims=True)

        @pl.when(p == 1)
        def _():
            u = sb - acc[...] / n
            n2v = jnp.sum(u * u, axis=1, keepdims=True)
            un = jnp.maximum(jnp.sqrt(n2v), _EPS)
            f = jnp.tanh(un) / un
            e0 = f * u
            en2 = jnp.sum(e0 * e0, axis=1, keepdims=True)
            en = jnp.maximum(jnp.sqrt(en2), _EPS)
            maxnorm = 1.0 - 4e-3
            scale = jnp.where(en > maxnorm, maxnorm / en, 1.0)
            o_ref[...] = e0 * scale

    return pl.pallas_call(
        body,
        grid=(2, g),
        in_specs=[pl.BlockSpec((blk, d), lambda p, i: (i, 0))] * 3,
        out_specs=pl.BlockSpec((blk, d), lambda p, i: (i, 0)),
        out_shape=jax.ShapeDtypeStruct((n, d), jnp.float32),
        scratch_shapes=[pltpu.VMEM((1, d), jnp.float32)],
    )(y1, y2, y3)


def kernel(x, edge_index, edge_weight):
    n, d = x.shape
    e = edge_index.shape[1]
    n2 = n // 2
    grp = _NS * _G16 * _CH
    maxg = -(-e // grp)

    dst = edge_index[0].astype(jnp.int32)
    src = edge_index[1].astype(jnp.int32)
    w32 = lax.bitcast_convert_type(edge_weight.astype(jnp.float32), jnp.int32)

    m0 = dst < n2
    c_id = jnp.where(m0, 0, 1).astype(jnp.int32)
    p0 = jnp.cumsum(m0.astype(jnp.int32))
    ar = jnp.arange(e, dtype=jnp.int32)

    nslot = 2 * _NS * maxg * _G16 * _CH
    sl_ar = jnp.arange(nslot, dtype=jnp.int32)
    lane = sl_ar % _CH
    t2 = sl_ar // _CH
    cc = t2 % _G16
    t3 = t2 // _G16
    gid = t3 % maxg
    t4 = t3 // maxg
    tile = t4 % _NS
    cfld = t4 // _NS
    qr = ((gid * _G16 + cc) * _CH + lane) * _NS + tile
    cnt0 = p0[-1]
    eidx0 = jnp.searchsorted(p0, qr + 1, side="left").astype(jnp.int32)
    eidx1 = jnp.searchsorted(ar + 1 - p0, qr + 1,
                             side="left").astype(jnp.int32)
    edge = jnp.where(cfld == 0, eidx0, eidx1)
    valid = jnp.where(cfld == 0, qr < cnt0, qr < e - cnt0)
    perm = jnp.where(valid, jnp.minimum(edge, e), e)

    srcp = jnp.concatenate([src, jnp.zeros((1,), jnp.int32)])[perm]
    dstl = jnp.concatenate([dst - c_id * n2, jnp.zeros((1,), jnp.int32)])[perm]
    wp = jnp.concatenate([w32, jnp.zeros((1,), jnp.int32)])[perm]
    shp = (2, _NS, maxg, _G16, _CH)
    pk = jnp.stack([srcp.reshape(shp), dstl.reshape(shp), wp.reshape(shp),
                    jnp.zeros(shp, jnp.int32)], axis=4)

    cnt1 = e - cnt0
    gpt0 = -(-(-(-cnt0 // _NS)) // (_G16 * _CH))
    gpt1 = -(-(-(-cnt1 // _NS)) // (_G16 * _CH))
    ng = jnp.zeros((16,), jnp.int32).at[0].set(gpt0).at[1].set(gpt1)

    xt = _pre_logmap(x)
    y1 = _spmm_sc(pk, ng, xt, n, d, maxg)
    y2 = _spmm_sc(pk, ng, y1, n, d, maxg)
    y3 = _spmm_sc(pk, ng, y2, n, d, maxg)
    return _post(y1, y2, y3, n, d)


# speedup vs baseline: 93.3435x; 93.3435x over previous
import dataclasses
import functools

import jax
import jax.numpy as jnp
from jax import lax
from jax.experimental import pallas as pl
from jax.experimental.pallas import tpu as pltpu
from jax.experimental.pallas import tpu_sc as plsc

_EPS = 1e-15
_NUM_LAYERS = 3
_NC = 2
_NS = 16
_L = 16
_CH = 64
_G8 = 8
_WB = 64


def _pre_logmap(x, hd):
    n, d = x.shape
    blk = 1000

    def body(x_ref, o_ref):
        xb = x_ref[...]
        pn = jnp.sqrt(jnp.sum(xb * xb, axis=1, keepdims=True))
        pn = jnp.maximum(pn, _EPS)
        z = jnp.clip(pn, -1.0 + 1e-7, 1.0 - 1e-7)
        at = 0.5 * (jnp.log1p(z) - jnp.log1p(-z))
        xt = xb / pn * at
        o_ref[0] = xt[:, :hd]
        o_ref[1] = xt[:, hd:]

    out = pl.pallas_call(
        body,
        grid=(n // blk,),
        in_specs=[pl.BlockSpec((blk, d), lambda i: (i, 0))],
        out_specs=pl.BlockSpec((2, blk, hd), lambda i: (0, i, 0)),
        out_shape=jax.ShapeDtypeStruct((2, n, hd), jnp.float32),
    )(x)
    return out.reshape(2 * n, hd)


def _spmm_sc(pk, xt, n, hd, nch):
    ngrp = nch // _G8
    rpt = (n // _NS) // 8 * 8
    tail = n - _NS * rpt
    wfull = rpt // _WB
    wrem = rpt - wfull * _WB

    mesh = plsc.VectorSubcoreMesh(core_axis_name="c", subcore_axis_name="s")
    cp = pltpu.CompilerParams()
    if "needs_layout_passes" in pltpu.CompilerParams.__dataclass_fields__:
        cp = dataclasses.replace(cp, needs_layout_passes=False)

    @functools.partial(
        pl.kernel,
        out_type=[jax.ShapeDtypeStruct((2 * n, hd), jnp.float32)] * 3,
        mesh=mesh,
        compiler_params=cp,
        scratch_types=[
            pltpu.VMEM_SHARED((n, hd), jnp.float32),
            pltpu.VMEM((2, _G8, 4, _CH), jnp.int32),
            pltpu.VMEM((4, _CH, hd), jnp.float32),
            pltpu.SemaphoreType.DMA((4,)),
            pltpu.SemaphoreType.DMA((4,)),
            pltpu.SemaphoreType.DMA((2,)),
        ],
    )
    def run(pk_hbm, xt_hbm, y1_hbm, y2_hbm, y3_hbm,
            acc, ibuf, rows, gsem, ssem, isem):
        c = lax.axis_index("c")
        s = lax.axis_index("s")
        base = s * rpt

        def idx_prefetch(q, g):
            pltpu.async_copy(pk_hbm.at[s, g], ibuf.at[q], isem.at[q])

        def wait_idx(q):
            pltpu.make_async_copy(pk_hbm.at[s, 0], ibuf.at[q],
                                  isem.at[q]).wait()

        def wait_scatter(q, cc, b):
            pltpu.make_async_copy(rows.at[b], acc.at[ibuf.at[q, cc, 2]],
                                  ssem.at[b]).wait()

        def layer(x_hbm, out_hbm):
            do_layer(x_hbm, out_hbm, idx_prefetch, wait_idx, wait_scatter)

        def do_layer(x_hbm, out_hbm, idx_prefetch, wait_idx, wait_scatter):
            @pl.loop(0, _WB)
            def _(i):
                for j in range(hd // _L):
                    rows[0, i, pl.ds(j * _L, _L)] = jnp.zeros(
                        (_L,), jnp.float32)

            for k in range(wfull):
                pltpu.sync_copy(rows.at[0],
                                acc.at[pl.ds(base + k * _WB, _WB)])
            if wrem:
                pltpu.sync_copy(rows.at[0, pl.ds(0, wrem)],
                                acc.at[pl.ds(base + wfull * _WB, wrem)])
            if tail:
                @pl.when(s == _NS - 1)
                def _():
                    pltpu.sync_copy(rows.at[0, pl.ds(0, tail)],
                                    acc.at[pl.ds(_NS * rpt, tail)])
            plsc.subcore_barrier()

            def issue_gather(q, cc, b):
                pltpu.async_copy(x_hbm.at[ibuf.at[q, cc, c]], rows.at[b],
                                 gsem.at[b])

            def wait_gather(q, cc, b):
                pltpu.make_async_copy(x_hbm.at[ibuf.at[q, cc, c]],
                                      rows.at[b], gsem.at[b]).wait()

            run_pipeline(issue_gather, wait_gather)

            plsc.subcore_barrier()

            obase = c * n + base
            for k in range(wfull):
                pltpu.sync_copy(acc.at[pl.ds(base + k * _WB, _WB)],
                                out_hbm.at[pl.ds(obase + k * _WB, _WB)])
            if wrem:
                pltpu.sync_copy(acc.at[pl.ds(base + wfull * _WB, wrem)],
                                out_hbm.at[pl.ds(obase + wfull * _WB, wrem)])
            if tail:
                @pl.when(s == _NS - 1)
                def _():
                    pltpu.sync_copy(
                        acc.at[pl.ds(_NS * rpt, tail)],
                        out_hbm.at[pl.ds(c * n + _NS * rpt, tail)])
            plsc.subcore_barrier()

        def run_pipeline(issue_gather, wait_gather):
            group_body(0, 1, issue_gather, wait_gather, first=True)

            @pl.loop(0, (ngrp - 2) // 2)
            def _(i):
                ga = 1 + 2 * i
                group_body(1, ga + 1, issue_gather, wait_gather)
                group_body(0, ga + 2, issue_gather, wait_gather)

            group_body(1, 0, issue_gather, wait_gather, last=True)
            for cc in range(4, _G8):
                wait_scatter(1, cc, cc % 4)

        def group_body(q, gnext, issue_gather, wait_gather,
                       first=False, last=False):
            if first:
                pltpu.sync_copy(pk_hbm.at[s, 0], ibuf.at[0])
                issue_gather(0, 0, 0)
                issue_gather(0, 1, 1)
            for cc in range(_G8):
                b = cc % 4
                if not (last and cc >= _G8 - 2):
                    if cc < _G8 - 2:
                        tq, tcc = q, cc + 2
                    else:
                        tq, tcc = 1 - q, cc + 2 - _G8
                    tb = (cc + 2) % 4
                    if not (first and cc < 2):
                        pq, pcc = (q, cc - 2) if cc >= 2 else (1 - q, cc + 6)
                        wait_scatter(pq, pcc, tb)
                    if cc == 3 and not last:
                        idx_prefetch(1 - q, gnext)
                    if tq != q and cc == _G8 - 2:
                        wait_idx(1 - q)
                    issue_gather(tq, tcc, tb)
                wait_gather(q, cc, b)

                @pl.loop(0, _CH, step=2)
                def _(e):
                    for u in range(2):
                        wv = plsc.bitcast(
                            plsc.load_gather(
                                ibuf.at[q, cc, 3],
                                [jnp.full((_L,), e + u, jnp.int32)]),
                            jnp.float32)
                        for j in range(hd // _L):
                            sl = pl.ds(j * _L, _L)
                            rows[b, e + u, sl] = rows[b, e + u, sl] * wv

                pltpu.async_copy(rows.at[b], acc.at[ibuf.at[q, cc, 2]],
                                 ssem.at[b], add=True)

        layer(xt_hbm, y1_hbm)
        layer(y1_hbm, y2_hbm)
        layer(y2_hbm, y3_hbm)

    return run(pk, xt)


def _post(y1, y2, y3, n, d):
    hd = d // 2
    blk = 1000
    g = n // blk

    def body(y1_ref, y2_ref, y3_ref, o_ref, acc):
        p = pl.program_id(0)
        i = pl.program_id(1)
        s0 = y1_ref[0] + y2_ref[0] + y3_ref[0]
        s1 = y1_ref[1] + y2_ref[1] + y3_ref[1]

        @pl.when(jnp.logical_and(p == 0, i == 0))
        def _():
            acc[...] = jnp.zeros_like(acc)

        @pl.when(p == 0)
        def _():
            acc[0:1, :] += jnp.sum(s0, axis=0, keepdims=True)
            acc[1:2, :] += jnp.sum(s1, axis=0, keepdims=True)

        @pl.when(p == 1)
        def _():
            u0 = s0 - acc[0:1, :] / n
            u1 = s1 - acc[1:2, :] / n
            n2 = (jnp.sum(u0 * u0, axis=1, keepdims=True)
                  + jnp.sum(u1 * u1, axis=1, keepdims=True))
            un = jnp.maximum(jnp.sqrt(n2), _EPS)
            f = jnp.tanh(un) / un
            e0 = f * u0
            e1 = f * u1
            en2 = (jnp.sum(e0 * e0, axis=1, keepdims=True)
                   + jnp.sum(e1 * e1, axis=1, keepdims=True))
            en = jnp.maximum(jnp.sqrt(en2), _EPS)
            maxnorm = 1.0 - 4e-3
            scale = jnp.where(en > maxnorm, maxnorm / en, 1.0)
            o_ref[:, :hd] = e0 * scale
            o_ref[:, hd:] = e1 * scale

    return pl.pallas_call(
        body,
        grid=(2, g),
        in_specs=[pl.BlockSpec((2, blk, hd), lambda p, i: (0, i, 0))] * 3,
        out_specs=pl.BlockSpec((blk, d), lambda p, i: (i, 0)),
        out_shape=jax.ShapeDtypeStruct((n, d), jnp.float32),
        scratch_shapes=[pltpu.VMEM((2, hd), jnp.float32)],
    )(y1, y2, y3)


def kernel(x, edge_index, edge_weight):
    n, d = x.shape
    hd = d // 2
    e = edge_index.shape[1]

    nch = -(-e // (_NS * _CH * 2 * _G8)) * 2 * _G8
    ept = nch * _CH
    e_pad = ept * _NS
    src = jnp.zeros((e_pad,), jnp.int32).at[:e].set(edge_index[1].astype(jnp.int32))
    dst = jnp.zeros((e_pad,), jnp.int32).at[:e].set(edge_index[0].astype(jnp.int32))
    w32 = lax.bitcast_convert_type(
        jnp.zeros((e_pad,), jnp.float32).at[:e].set(edge_weight), jnp.int32)
    pk = jnp.stack([src, src + n, dst, w32], axis=0)
    pk = pk.reshape(4, _NS, nch // _G8, _G8, _CH).transpose(1, 2, 3, 0, 4)

    xt = _pre_logmap(x, hd)
    y1, y2, y3 = _spmm_sc(pk, xt, n, hd, nch)
    return _post(y1.reshape(2, n, hd), y2.reshape(2, n, hd),
                 y3.reshape(2, n, hd), n, d)


# speedup vs baseline: 93.7617x; 1.0045x over previous
import dataclasses
import functools

import jax
import jax.numpy as jnp
from jax import lax
from jax.experimental import pallas as pl
from jax.experimental.pallas import tpu as pltpu
from jax.experimental.pallas import tpu_sc as plsc

_EPS = 1e-15
_NUM_LAYERS = 3
_NC = 2
_NS = 16
_L = 16
_CH = 64
_G8 = 5
_WB = 64


def _pre_logmap(x, hd):
    n, d = x.shape
    blk = 1000

    def body(x_ref, o_ref):
        xb = x_ref[...]
        pn = jnp.sqrt(jnp.sum(xb * xb, axis=1, keepdims=True))
        pn = jnp.maximum(pn, _EPS)
        z = jnp.clip(pn, -1.0 + 1e-7, 1.0 - 1e-7)
        at = 0.5 * (jnp.log1p(z) - jnp.log1p(-z))
        xt = xb / pn * at
        o_ref[0] = xt[:, :hd]
        o_ref[1] = xt[:, hd:]

    out = pl.pallas_call(
        body,
        grid=(n // blk,),
        in_specs=[pl.BlockSpec((blk, d), lambda i: (i, 0))],
        out_specs=pl.BlockSpec((2, blk, hd), lambda i: (0, i, 0)),
        out_shape=jax.ShapeDtypeStruct((2, n, hd), jnp.float32),
    )(x)
    return out.reshape(2 * n, hd)


def _spmm_sc(pk, xt, n, hd, nch):
    ngrp = nch // _G8
    rpt = (n // _NS) // 8 * 8
    tail = n - _NS * rpt
    wfull = rpt // _WB
    wrem = rpt - wfull * _WB

    mesh = plsc.VectorSubcoreMesh(core_axis_name="c", subcore_axis_name="s")
    cp = pltpu.CompilerParams()
    if "needs_layout_passes" in pltpu.CompilerParams.__dataclass_fields__:
        cp = dataclasses.replace(cp, needs_layout_passes=False)

    @functools.partial(
        pl.kernel,
        out_type=[jax.ShapeDtypeStruct((2 * n, hd), jnp.float32)] * 3,
        mesh=mesh,
        compiler_params=cp,
        scratch_types=[
            pltpu.VMEM_SHARED((n, hd), jnp.float32),
            pltpu.VMEM((3, _G8, 4, _CH), jnp.int32),
            pltpu.VMEM((5, _CH, hd), jnp.float32),
            pltpu.SemaphoreType.DMA((5,)),
            pltpu.SemaphoreType.DMA((5,)),
            pltpu.SemaphoreType.DMA((3,)),
        ],
    )
    def run(pk_hbm, xt_hbm, y1_hbm, y2_hbm, y3_hbm,
            acc, ibuf, rows, gsem, ssem, isem):
        c = lax.axis_index("c")
        s = lax.axis_index("s")
        base = s * rpt

        def idx_prefetch(q, g):
            pltpu.async_copy(pk_hbm.at[s, g], ibuf.at[q], isem.at[q])

        def wait_idx(q):
            pltpu.make_async_copy(pk_hbm.at[s, 0], ibuf.at[q],
                                  isem.at[q]).wait()

        def wait_scatter(q, cc, b):
            pltpu.make_async_copy(rows.at[b], acc.at[ibuf.at[q, cc, 2]],
                                  ssem.at[b]).wait()

        def layer(x_hbm, out_hbm):
            do_layer(x_hbm, out_hbm, idx_prefetch, wait_idx, wait_scatter)

        def do_layer(x_hbm, out_hbm, idx_prefetch, wait_idx, wait_scatter):
            @pl.loop(0, _WB)
            def _(i):
                for j in range(hd // _L):
                    rows[0, i, pl.ds(j * _L, _L)] = jnp.zeros(
                        (_L,), jnp.float32)

            for k in range(wfull):
                pltpu.sync_copy(rows.at[0],
                                acc.at[pl.ds(base + k * _WB, _WB)])
            if wrem:
                pltpu.sync_copy(rows.at[0, pl.ds(0, wrem)],
                                acc.at[pl.ds(base + wfull * _WB, wrem)])
            if tail:
                @pl.when(s == _NS - 1)
                def _():
                    pltpu.sync_copy(rows.at[0, pl.ds(0, tail)],
                                    acc.at[pl.ds(_NS * rpt, tail)])
            plsc.subcore_barrier()

            def issue_gather(q, cc, b):
                pltpu.async_copy(x_hbm.at[ibuf.at[q, cc, c]], rows.at[b],
                                 gsem.at[b])

            def wait_gather(q, cc, b):
                pltpu.make_async_copy(x_hbm.at[ibuf.at[q, cc, c]],
                                      rows.at[b], gsem.at[b]).wait()

            run_pipeline(issue_gather, wait_gather)

            plsc.subcore_barrier()

            obase = c * n + base
            for k in range(wfull):
                pltpu.sync_copy(acc.at[pl.ds(base + k * _WB, _WB)],
                                out_hbm.at[pl.ds(obase + k * _WB, _WB)])
            if wrem:
                pltpu.sync_copy(acc.at[pl.ds(base + wfull * _WB, wrem)],
                                out_hbm.at[pl.ds(obase + wfull * _WB, wrem)])
            if tail:
                @pl.when(s == _NS - 1)
                def _():
                    pltpu.sync_copy(
                        acc.at[pl.ds(_NS * rpt, tail)],
                        out_hbm.at[pl.ds(c * n + _NS * rpt, tail)])
            plsc.subcore_barrier()

        def run_pipeline(issue_gather, wait_gather):
            group_body(0, jnp.int32(2), issue_gather, wait_gather,
                       first=True)

            @pl.loop(0, (ngrp - 2) // 3)
            def _(i):
                ga = 1 + 3 * i
                group_body(1, ga + 2, issue_gather, wait_gather)
                group_body(2, ga + 3, issue_gather, wait_gather)
                group_body(0, ga + 4, issue_gather, wait_gather)

            group_body((ngrp - 1) % 3, 0, issue_gather, wait_gather,
                       last=True)
            for cc in range(_G8):
                wait_scatter((ngrp - 1) % 3, cc, cc)

        def group_body(q, gnext, issue_gather, wait_gather,
                       first=False, last=False):
            qn = (q + 1) % 3
            qp = (q + 2) % 3
            if first:
                pltpu.sync_copy(pk_hbm.at[s, 0], ibuf.at[0])
                idx_prefetch(1, jnp.int32(1))
                issue_gather(0, 0, 0)
                issue_gather(0, 1, 1)
                issue_gather(0, 2, 2)
            for cc in range(_G8):
                if not (last and cc >= 2):
                    if cc < 2:
                        tq, tcc = q, cc + 3
                    else:
                        tq, tcc = qn, cc - 2
                    tb = (cc + 3) % 5
                    if not (first and cc < 2):
                        pq, pcc = (q, cc - 2) if cc >= 2 else (qp, cc + 3)
                        wait_scatter(pq, pcc, tb)
                    if cc == 2:
                        @pl.when(gnext < ngrp)
                        def _():
                            idx_prefetch(qp, gnext)
                        wait_idx(qn)
                    issue_gather(tq, tcc, tb)
                wait_gather(q, cc, cc)

                @pl.loop(0, _CH, step=2)
                def _(e):
                    for u in range(2):
                        wv = plsc.bitcast(
                            plsc.load_gather(
                                ibuf.at[q, cc, 3],
                                [jnp.full((_L,), e + u, jnp.int32)]),
                            jnp.float32)
                        for j in range(hd // _L):
                            sl = pl.ds(j * _L, _L)
                            rows[cc, e + u, sl] = rows[cc, e + u, sl] * wv

                pltpu.async_copy(rows.at[cc], acc.at[ibuf.at[q, cc, 2]],
                                 ssem.at[cc], add=True)

        layer(xt_hbm, y1_hbm)
        layer(y1_hbm, y2_hbm)
        layer(y2_hbm, y3_hbm)

    return run(pk, xt)


def _post(y1, y2, y3, n, d):
    hd = d // 2
    blk = 1000
    g = n // blk

    def body(y1_ref, y2_ref, y3_ref, o_ref, acc):
        p = pl.program_id(0)
        i = pl.program_id(1)
        s0 = y1_ref[0] + y2_ref[0] + y3_ref[0]
        s1 = y1_ref[1] + y2_ref[1] + y3_ref[1]

        @pl.when(jnp.logical_and(p == 0, i == 0))
        def _():
            acc[...] = jnp.zeros_like(acc)

        @pl.when(p == 0)
        def _():
            acc[0:1, :] += jnp.sum(s0, axis=0, keepdims=True)
            acc[1:2, :] += jnp.sum(s1, axis=0, keepdims=True)

        @pl.when(p == 1)
        def _():
            u0 = s0 - acc[0:1, :] / n
            u1 = s1 - acc[1:2, :] / n
            n2 = (jnp.sum(u0 * u0, axis=1, keepdims=True)
                  + jnp.sum(u1 * u1, axis=1, keepdims=True))
            un = jnp.maximum(jnp.sqrt(n2), _EPS)
            f = jnp.tanh(un) / un
            e0 = f * u0
            e1 = f * u1
            en2 = (jnp.sum(e0 * e0, axis=1, keepdims=True)
                   + jnp.sum(e1 * e1, axis=1, keepdims=True))
            en = jnp.maximum(jnp.sqrt(en2), _EPS)
            maxnorm = 1.0 - 4e-3
            scale = jnp.where(en > maxnorm, maxnorm / en, 1.0)
            o_ref[:, :hd] = e0 * scale
            o_ref[:, hd:] = e1 * scale

    return pl.pallas_call(
        body,
        grid=(2, g),
        in_specs=[pl.BlockSpec((2, blk, hd), lambda p, i: (0, i, 0))] * 3,
        out_specs=pl.BlockSpec((blk, d), lambda p, i: (i, 0)),
        out_shape=jax.ShapeDtypeStruct((n, d), jnp.float32),
        scratch_shapes=[pltpu.VMEM((2, hd), jnp.float32)],
    )(y1, y2, y3)


def kernel(x, edge_index, edge_weight):
    n, d = x.shape
    hd = d // 2
    e = edge_index.shape[1]

    nch = -(-e // (_NS * _CH * _G8)) * _G8
    while ((nch // _G8) - 2) % 3 or (nch // _G8) < 5:
        nch += _G8
    ept = nch * _CH
    e_pad = ept * _NS
    src = jnp.zeros((e_pad,), jnp.int32).at[:e].set(edge_index[1].astype(jnp.int32))
    dst = jnp.zeros((e_pad,), jnp.int32).at[:e].set(edge_index[0].astype(jnp.int32))
    w32 = lax.bitcast_convert_type(
        jnp.zeros((e_pad,), jnp.float32).at[:e].set(edge_weight), jnp.int32)
    pk = jnp.stack([src, src + n, dst, w32], axis=0)
    pk = pk.reshape(4, _NS, nch // _G8, _G8, _CH).transpose(1, 2, 3, 0, 4)

    xt = _pre_logmap(x, hd)
    y1, y2, y3 = _spmm_sc(pk, xt, n, hd, nch)
    return _post(y1.reshape(2, n, hd), y2.reshape(2, n, hd),
                 y3.reshape(2, n, hd), n, d)
